# baseline (speedup 1.0000x reference)
import functools

import jax
import jax.numpy as jnp
from jax import lax
from jax.experimental import pallas as pl
from jax.experimental.pallas import tpu as pltpu
from jax.experimental.pallas import tpu_sc as plsc

_B, _S = 4096, 26
_F = 32
_TOTAL = _B * _S
_NC, _NS = 2, 16
_NW = _NC * _NS
_PER_W = _TOTAL // _NW
_CHUNK = 128
_NCHUNK = _PER_W // _CHUNK
_QROWS = 250000

_mesh = plsc.VectorSubcoreMesh(core_axis_name="c", subcore_axis_name="s")


@functools.partial(
    pl.kernel,
    out_type=jax.ShapeDtypeStruct((_S, _F, _B), jnp.float32),
    mesh=_mesh,
    scratch_types=[
        pltpu.VMEM((_NCHUNK, _CHUNK), jnp.int32),
        pltpu.VMEM((_NCHUNK, _CHUNK), jnp.int32),
        pltpu.VMEM((_NCHUNK, _CHUNK), jnp.int32),
        pltpu.VMEM((2, _CHUNK, _CHUNK), jnp.float32),
        pltpu.VMEM((2, _F, _CHUNK), jnp.float32),
        pltpu.SemaphoreType.DMA,
        pltpu.SemaphoreType.DMA,
    ],
    compiler_params=pltpu.CompilerParams(needs_layout_passes=False),
)
def _gather_kernel(
    idx_hbm, table_hbm, out_hbm, idx_v, g_v, q_v, quads, rowsT, gsem, osem
):
    wid = lax.axis_index("s") * _NC + lax.axis_index("c")
    pltpu.sync_copy(idx_hbm.at[wid], idx_v)
    for j in range(_NCHUNK):
        for bg in range(_CHUNK // 16):
            v = idx_v[j, pl.ds(bg * 16, 16)]
            g_v[j, pl.ds(bg * 16, 16)] = v >> 2
            q_v[j, pl.ds(bg * 16, 16)] = (v & 3) << 5
    for j in range(2):
        pltpu.async_copy(table_hbm.at[g_v.at[j]], quads.at[j & 1], gsem)

    lanes = lax.iota(jnp.int32, 16)

    def per_chunk(j, carry):
        pltpu.make_async_copy(
            table_hbm.at[g_v.at[0]], quads.at[0], gsem
        ).wait()
        qbuf = quads.at[j & 1]
        tbuf = rowsT.at[j & 1]

        @pl.when(j >= 2)
        def _():
            pltpu.make_async_copy(
                tbuf, out_hbm.at[0, :, pl.ds(0, _CHUNK)], osem
            ).wait()

        for bg in range(_CHUNK // 16):
            b_ids = lanes + bg * 16
            qb = q_v[j, pl.ds(bg * 16, 16)]
            for f in range(_F):
                v = plsc.load_gather(qbuf, [b_ids, qb + f])
                tbuf[f, pl.ds(bg * 16, 16)] = v

        @pl.when(j + 2 < _NCHUNK)
        def _():
            pltpu.async_copy(table_hbm.at[g_v.at[j + 2]], qbuf, gsem)

        u = wid * _NCHUNK + j
        s = u >> 5
        bblk = u & 31
        pltpu.async_copy(
            tbuf, out_hbm.at[s, :, pl.ds(bblk * _CHUNK, _CHUNK)], osem
        )
        return carry

    lax.fori_loop(0, _NCHUNK, per_chunk, 0)
    for _ in range(2):
        pltpu.make_async_copy(
            rowsT.at[0], out_hbm.at[0, :, pl.ds(0, _CHUNK)], osem
        ).wait()


_STRIPE = 512
_NSTRIPE = 1953
_TAIL = 64


@functools.partial(
    pl.kernel,
    out_type=jax.ShapeDtypeStruct((_QROWS, _CHUNK), jnp.float32),
    mesh=_mesh,
    scratch_types=[
        pltpu.VMEM((2, _F, _STRIPE), jnp.float32),
        pltpu.VMEM((2, _STRIPE // 4, _CHUNK), jnp.float32),
        pltpu.VMEM((_TAIL, _F), jnp.float32),
        pltpu.SemaphoreType.DMA,
        pltpu.SemaphoreType.DMA,
    ],
    compiler_params=pltpu.CompilerParams(needs_layout_passes=False),
)
def _transpose_kernel(tabT_hbm, tail_hbm, outq_hbm, stage, otile, tailv, isem, osem):
    wid = lax.axis_index("s") * _NC + lax.axis_index("c")
    n = (_NSTRIPE - wid + 31) >> 5
    lanes = lax.iota(jnp.int32, 16)

    pltpu.async_copy(
        tabT_hbm.at[:, pl.ds(wid * _STRIPE, _STRIPE)], stage.at[0], isem
    )

    def per_stripe(k, carry):
        s = wid + k * _NW
        pltpu.make_async_copy(
            tabT_hbm.at[:, pl.ds(0, _STRIPE)], stage.at[0], isem
        ).wait()
        sbuf = stage.at[k & 1]
        obuf = otile.at[k & 1]

        @pl.when(k + 1 < n)
        def _():
            s2 = s + _NW
            pltpu.async_copy(
                tabT_hbm.at[:, pl.ds(s2 * _STRIPE, _STRIPE)],
                stage.at[(k + 1) & 1],
                isem,
            )

        @pl.when(k >= 2)
        def _():
            pltpu.make_async_copy(
                obuf, outq_hbm.at[pl.ds(0, _STRIPE // 4)], osem
            ).wait()

        for dgrp in range(_STRIPE // 16):
            di = dgrp * 16 + lanes
            dg_ids = di >> 2
            base = (di & 3) << 5
            for f in range(_F):
                v = sbuf[f, pl.ds(dgrp * 16, 16)]
                plsc.store_scatter(obuf, [dg_ids, base + f], v)

        pltpu.async_copy(
            obuf, outq_hbm.at[pl.ds(s * (_STRIPE // 4), _STRIPE // 4)], osem
        )
        return carry

    lax.fori_loop(0, n, per_stripe, 0)
    for _ in range(2):
        pltpu.make_async_copy(
            otile.at[0], outq_hbm.at[pl.ds(0, _STRIPE // 4)], osem
        ).wait()

    @pl.when(wid == 0)
    def _():
        pltpu.sync_copy(tail_hbm, tailv)
        for r in range(_TAIL):
            for h in range(2):
                v = tailv[r, pl.ds(h * 16, 16)]
                otile[0, r >> 2, pl.ds((r & 3) * _F + h * 16, 16)] = v
        pltpu.sync_copy(
            otile.at[0, pl.ds(0, _TAIL // 4)],
            outq_hbm.at[pl.ds((_NSTRIPE * _STRIPE) // 4, _TAIL // 4)],
        )


def kernel(inputs, embedding):
    idx = inputs.T.reshape(_NW, _NCHUNK, _CHUNK)
    table_q = _transpose_kernel(embedding.T, embedding[_NSTRIPE * _STRIPE:])
    out = _gather_kernel(idx, table_q)
    return out.transpose(2, 0, 1)


# speedup vs baseline: 1.2620x; 1.2620x over previous
import functools

import jax
import jax.numpy as jnp
from jax import lax
from jax.experimental import pallas as pl
from jax.experimental.pallas import tpu as pltpu
from jax.experimental.pallas import tpu_sc as plsc

_B, _S = 4096, 26
_F = 32
_TOTAL = _B * _S
_NC, _NS = 2, 16
_NW = _NC * _NS
_PER_W = _TOTAL // _NW
_CHUNK = 128
_NCHUNK = _PER_W // _CHUNK
_QROWS = 250000

_mesh = plsc.VectorSubcoreMesh(core_axis_name="c", subcore_axis_name="s")


@functools.partial(
    pl.kernel,
    out_type=jax.ShapeDtypeStruct((_S, _F, _B), jnp.float32),
    mesh=_mesh,
    scratch_types=[
        pltpu.VMEM((_NCHUNK, _CHUNK), jnp.int32),
        pltpu.VMEM((_NCHUNK, _CHUNK), jnp.int32),
        pltpu.VMEM((_NCHUNK, _CHUNK), jnp.int32),
        pltpu.VMEM((2, _CHUNK, _CHUNK), jnp.float32),
        pltpu.VMEM((2, _F, _CHUNK), jnp.float32),
        pltpu.SemaphoreType.DMA,
        pltpu.SemaphoreType.DMA,
    ],
    compiler_params=pltpu.CompilerParams(needs_layout_passes=False),
)
def _gather_kernel(
    idx_hbm, table_hbm, out_hbm, idx_v, g_v, q_v, quads, rowsT, gsem, osem
):
    wid = lax.axis_index("s") * _NC + lax.axis_index("c")
    pltpu.sync_copy(idx_hbm.at[wid], idx_v)
    for j in range(_NCHUNK):
        for bg in range(_CHUNK // 16):
            v = idx_v[j, pl.ds(bg * 16, 16)]
            g_v[j, pl.ds(bg * 16, 16)] = v >> 2
            q_v[j, pl.ds(bg * 16, 16)] = (v & 3) << 5
    for j in range(2):
        pltpu.async_copy(table_hbm.at[g_v.at[j]], quads.at[j & 1], gsem)

    lanes = lax.iota(jnp.int32, 16)

    def per_chunk(j, carry):
        pltpu.make_async_copy(
            table_hbm.at[g_v.at[0]], quads.at[0], gsem
        ).wait()
        qbuf = quads.at[j & 1]
        tbuf = rowsT.at[j & 1]

        @pl.when(j >= 2)
        def _():
            pltpu.make_async_copy(
                tbuf, out_hbm.at[0, :, pl.ds(0, _CHUNK)], osem
            ).wait()

        @plsc.parallel_loop(0, _CHUNK // 16, step=1, unroll=4)
        def _eloop(bg):
            b_ids = lanes + bg * 16
            qb = q_v[j, pl.ds(bg * 16, 16)]
            for f in range(_F):
                v = plsc.load_gather(qbuf, [b_ids, qb + f])
                tbuf[f, pl.ds(bg * 16, 16)] = v

        @pl.when(j + 2 < _NCHUNK)
        def _():
            pltpu.async_copy(table_hbm.at[g_v.at[j + 2]], qbuf, gsem)

        u = wid * _NCHUNK + j
        s = u >> 5
        bblk = u & 31
        pltpu.async_copy(
            tbuf, out_hbm.at[s, :, pl.ds(bblk * _CHUNK, _CHUNK)], osem
        )
        return carry

    lax.fori_loop(0, _NCHUNK, per_chunk, 0)
    for _ in range(2):
        pltpu.make_async_copy(
            rowsT.at[0], out_hbm.at[0, :, pl.ds(0, _CHUNK)], osem
        ).wait()


_STRIPE = 512
_NSTRIPE = 1953
_TAIL = 64


@functools.partial(
    pl.kernel,
    out_type=jax.ShapeDtypeStruct((_QROWS, _CHUNK), jnp.float32),
    mesh=_mesh,
    scratch_types=[
        pltpu.VMEM((2, _F, _STRIPE), jnp.float32),
        pltpu.VMEM((2, _STRIPE // 4, _CHUNK), jnp.float32),
        pltpu.VMEM((_TAIL, _F), jnp.float32),
        pltpu.SemaphoreType.DMA,
        pltpu.SemaphoreType.DMA,
    ],
    compiler_params=pltpu.CompilerParams(needs_layout_passes=False),
)
def _transpose_kernel(tabT_hbm, tail_hbm, outq_hbm, stage, otile, tailv, isem, osem):
    wid = lax.axis_index("s") * _NC + lax.axis_index("c")
    n = (_NSTRIPE - wid + 31) >> 5
    lanes = lax.iota(jnp.int32, 16)

    pltpu.async_copy(
        tabT_hbm.at[:, pl.ds(wid * _STRIPE, _STRIPE)], stage.at[0], isem
    )

    def per_stripe(k, carry):
        s = wid + k * _NW
        pltpu.make_async_copy(
            tabT_hbm.at[:, pl.ds(0, _STRIPE)], stage.at[0], isem
        ).wait()
        sbuf = stage.at[k & 1]
        obuf = otile.at[k & 1]

        @pl.when(k + 1 < n)
        def _():
            s2 = s + _NW
            pltpu.async_copy(
                tabT_hbm.at[:, pl.ds(s2 * _STRIPE, _STRIPE)],
                stage.at[(k + 1) & 1],
                isem,
            )

        @pl.when(k >= 2)
        def _():
            pltpu.make_async_copy(
                obuf, outq_hbm.at[pl.ds(0, _STRIPE // 4)], osem
            ).wait()

        @plsc.parallel_loop(0, _STRIPE // 16, step=1, unroll=4)
        def _tloop(dgrp):
            di = dgrp * 16 + lanes
            dg_ids = di >> 2
            base = (di & 3) << 5
            for f in range(_F):
                v = sbuf[f, pl.ds(dgrp * 16, 16)]
                plsc.store_scatter(obuf, [dg_ids, base + f], v)

        pltpu.async_copy(
            obuf, outq_hbm.at[pl.ds(s * (_STRIPE // 4), _STRIPE // 4)], osem
        )
        return carry

    lax.fori_loop(0, n, per_stripe, 0)
    for _ in range(2):
        pltpu.make_async_copy(
            otile.at[0], outq_hbm.at[pl.ds(0, _STRIPE // 4)], osem
        ).wait()

    @pl.when(wid == 0)
    def _():
        pltpu.sync_copy(tail_hbm, tailv)
        for r in range(_TAIL):
            for h in range(2):
                v = tailv[r, pl.ds(h * 16, 16)]
                otile[0, r >> 2, pl.ds((r & 3) * _F + h * 16, 16)] = v
        pltpu.sync_copy(
            otile.at[0, pl.ds(0, _TAIL // 4)],
            outq_hbm.at[pl.ds((_NSTRIPE * _STRIPE) // 4, _TAIL // 4)],
        )


def kernel(inputs, embedding):
    idx = inputs.T.reshape(_NW, _NCHUNK, _CHUNK)
    table_q = _transpose_kernel(embedding.T, embedding[_NSTRIPE * _STRIPE:])
    out = _gather_kernel(idx, table_q)
    return out.transpose(2, 0, 1)


# speedup vs baseline: 1.2694x; 1.0059x over previous
import functools

import jax
import jax.numpy as jnp
from jax import lax
from jax.experimental import pallas as pl
from jax.experimental.pallas import tpu as pltpu
from jax.experimental.pallas import tpu_sc as plsc

_B, _S = 4096, 26
_F = 32
_TOTAL = _B * _S
_NC, _NS = 2, 16
_NW = _NC * _NS
_PER_W = _TOTAL // _NW
_CHUNK = 128
_NCHUNK = _PER_W // _CHUNK

_mesh = plsc.VectorSubcoreMesh(core_axis_name="c", subcore_axis_name="s")


@functools.partial(
    pl.kernel,
    out_type=jax.ShapeDtypeStruct((_TOTAL, _F), jnp.float32),
    mesh=_mesh,
    scratch_types=[
        pltpu.VMEM((_NCHUNK, _CHUNK), jnp.int32),
        pltpu.VMEM((_PER_W, _F), jnp.float32),
        pltpu.SemaphoreType.DMA,
    ],
    compiler_params=pltpu.CompilerParams(use_tc_tiling_on_sc=False),
)
def _gather_kernel(idx_hbm, table_hbm, out_hbm, idx_v, rows_v, sem):
    wid = lax.axis_index("s") * _NC + lax.axis_index("c")
    pltpu.sync_copy(idx_hbm.at[wid], idx_v)
    copies = []
    for j in range(_NCHUNK):
        copies.append(
            pltpu.async_copy(
                table_hbm.at[idx_v.at[j]],
                rows_v.at[pl.ds(j * _CHUNK, _CHUNK)],
                sem,
            )
        )
    for c in copies:
        c.wait()
    pltpu.sync_copy(rows_v, out_hbm.at[pl.ds(wid * _PER_W, _PER_W)])


def kernel(inputs, embedding):
    idx = inputs.reshape(_NW, _NCHUNK, _CHUNK)
    out = _gather_kernel(idx, embedding)
    return out.reshape(_B, _S, _F)
